# baseline (speedup 1.0000x reference)
import functools

import jax
import jax.numpy as jnp
from jax import lax
from jax.experimental import pallas as pl
from jax.experimental.pallas import tpu as pltpu
from jax.experimental.pallas import tpu_sc as plsc

LN2 = 0.6931471805599453


def _sc_gather(table, idx, chunk):
    V, D = table.shape
    B = idx.shape[0]
    mesh = plsc.VectorSubcoreMesh(core_axis_name="c", subcore_axis_name="s")
    nw = 32
    b_per_w = B // nw
    n_iter = b_per_w // chunk
    assert b_per_w % chunk == 0 and chunk % 8 == 0 and n_iter >= 4
    nbuf = 4
    lag = 2

    @functools.partial(
        pl.kernel,
        mesh=mesh,
        out_type=jax.ShapeDtypeStruct((B, D), jnp.float32),
        scratch_types=[
            pltpu.VMEM((b_per_w,), jnp.int32),
        ]
        + [pltpu.VMEM((chunk, D), jnp.float32)] * nbuf
        + [pltpu.SemaphoreType.DMA] * (2 * nbuf),
    )
    def gather_kernel(table_hbm, idx_hbm, out_hbm, idx_v, *bufs_sems):
        bufs = bufs_sems[:nbuf]
        gsem = bufs_sems[nbuf:2 * nbuf]
        wsem = bufs_sems[2 * nbuf:]
        wid = lax.axis_index("s") * 2 + lax.axis_index("c")
        base = wid * b_per_w

        pltpu.sync_copy(idx_hbm.at[pl.ds(base, b_per_w)], idx_v)

        g = {}
        w = {}

        def start_wb(jt):
            g[jt].wait()
            w[jt] = pltpu.async_copy(
                bufs[jt % nbuf],
                out_hbm.at[pl.ds(base + jt * chunk, chunk)],
                wsem[jt % nbuf])

        for it in range(n_iter):
            b = it % nbuf
            if it >= nbuf:
                w[it - nbuf].wait()
            g[it] = pltpu.async_copy(
                table_hbm.at[idx_v.at[pl.ds(it * chunk, chunk)]],
                bufs[b], gsem[b])
            if it >= lag:
                start_wb(it - lag)
        for jt in range(n_iter - lag, n_iter):
            start_wb(jt)
        for jt in range(max(0, n_iter - nbuf), n_iter):
            w[jt].wait()

    return gather_kernel(table, idx)


def _mlp_body(*refs):
    if len(refs) == 11:
        (h1_ref, h2_ref, ea_ref, w1a_ref, w1b_ref, w1c_ref, b1_ref,
         w2_ref, b2_ref, _prev_ref, o_ref) = refs
    else:
        (h1_ref, h2_ref, ea_ref, w1a_ref, w1b_ref, w1c_ref, b1_ref,
         w2_ref, b2_ref, o_ref) = refs
    bf = jnp.bfloat16
    x = jnp.dot(h1_ref[...].astype(bf), w1a_ref[...],
                preferred_element_type=jnp.float32)
    x += jnp.dot(h2_ref[...].astype(bf), w1b_ref[...],
                 preferred_element_type=jnp.float32)
    x += jnp.dot(ea_ref[...].astype(bf), w1c_ref[...],
                 preferred_element_type=jnp.float32)
    x += b1_ref[...]
    xb = x.astype(bf)
    y = (jnp.maximum(xb, 0) + jnp.log1p(jnp.exp(-jnp.abs(xb)))
         - jnp.asarray(LN2, bf))
    o_ref[...] = (
        jnp.dot(y, w2_ref[...], preferred_element_type=jnp.float32)
        + b2_ref[...]
    )


def _tc_mlp_chunk(hh, edge_attr, wb, prev_out, k, n_chunks, block):
    w1a, w1b, w1c, b1r, w2b, b2r = wb
    E = edge_attr.shape[0]
    ec = E // n_chunks
    d_feat = hh.shape[1]
    d_edge = edge_attr.shape[1]
    two_c = w1a.shape[1]
    C = w2b.shape[1]
    nb = ec // block
    koff = k * nb

    in_specs = [
        pl.BlockSpec((block, d_feat), lambda i: (i, 0)),
        pl.BlockSpec((block, d_feat), lambda i: (i + nb, 0)),
        pl.BlockSpec((block, d_edge), lambda i: (koff + i, 0)),
        pl.BlockSpec((d_feat, two_c), lambda i: (0, 0)),
        pl.BlockSpec((d_feat, two_c), lambda i: (0, 0)),
        pl.BlockSpec((d_edge, two_c), lambda i: (0, 0)),
        pl.BlockSpec((1, two_c), lambda i: (0, 0)),
        pl.BlockSpec((two_c, C), lambda i: (0, 0)),
        pl.BlockSpec((1, C), lambda i: (0, 0)),
    ]
    args = [hh, hh, edge_attr, w1a, w1b, w1c, b1r, w2b, b2r]
    aliases = {}
    if prev_out is not None:
        in_specs.append(pl.BlockSpec((8, C), lambda i: (0, 0)))
        args.append(prev_out)
        aliases = {9: 0}

    return pl.pallas_call(
        _mlp_body,
        grid=(nb,),
        in_specs=in_specs,
        out_specs=pl.BlockSpec((block, C), lambda i: (koff + i, 0)),
        out_shape=jax.ShapeDtypeStruct((E, C), jnp.float32),
        input_output_aliases=aliases,
    )(*args)


def kernel(h, edge_attr, edge_index, W1, b1, W2, b2):
    E = edge_attr.shape[0]
    d_feat = h.shape[1]
    d_edge = edge_attr.shape[1]
    two_c = W1.shape[1]
    C = W2.shape[1]
    n_chunks = 4
    ec = E // n_chunks

    ei = edge_index.astype(jnp.int32)
    idx_all = jnp.concatenate(
        [ei[0].reshape(n_chunks, ec), ei[1].reshape(n_chunks, ec)], axis=1)

    wb = (
        W1[:d_feat].astype(jnp.bfloat16),
        W1[d_feat:2 * d_feat].astype(jnp.bfloat16),
        W1[2 * d_feat:].astype(jnp.bfloat16),
        b1.reshape(1, two_c),
        W2.astype(jnp.bfloat16),
        b2.reshape(1, C),
    )

    out = None
    for k in range(n_chunks):
        hh_k = _sc_gather(h, idx_all[k], chunk=200)
        out = _tc_mlp_chunk(edge_attr=edge_attr, hh=hh_k, wb=wb,
                            prev_out=out, k=k, n_chunks=n_chunks, block=2000)
    return out


# speedup vs baseline: 1.0585x; 1.0585x over previous
import functools

import jax
import jax.numpy as jnp
from jax import lax
from jax.experimental import pallas as pl
from jax.experimental.pallas import tpu as pltpu
from jax.experimental.pallas import tpu_sc as plsc

LN2 = 0.6931471805599453


def _sc_gather(table, idx, chunk):
    V, D = table.shape
    B = idx.shape[0]
    mesh = plsc.VectorSubcoreMesh(core_axis_name="c", subcore_axis_name="s")
    nw = 32
    b_per_w = B // nw
    n_iter = b_per_w // chunk
    assert b_per_w % chunk == 0 and chunk % 8 == 0 and n_iter >= 4
    nbuf = 4
    lag = 2

    @functools.partial(
        pl.kernel,
        mesh=mesh,
        out_type=jax.ShapeDtypeStruct((B, D), jnp.float32),
        scratch_types=[
            pltpu.VMEM((b_per_w,), jnp.int32),
        ]
        + [pltpu.VMEM((chunk, D), jnp.float32)] * nbuf
        + [pltpu.SemaphoreType.DMA] * (2 * nbuf),
    )
    def gather_kernel(table_hbm, idx_hbm, out_hbm, idx_v, *bufs_sems):
        bufs = bufs_sems[:nbuf]
        gsem = bufs_sems[nbuf:2 * nbuf]
        wsem = bufs_sems[2 * nbuf:]
        wid = lax.axis_index("s") * 2 + lax.axis_index("c")
        base = wid * b_per_w

        pltpu.sync_copy(idx_hbm.at[pl.ds(base, b_per_w)], idx_v)

        g = {}
        w = {}

        def start_wb(jt):
            g[jt].wait()
            w[jt] = pltpu.async_copy(
                bufs[jt % nbuf],
                out_hbm.at[pl.ds(base + jt * chunk, chunk)],
                wsem[jt % nbuf])

        for it in range(n_iter):
            b = it % nbuf
            if it >= nbuf:
                w[it - nbuf].wait()
            g[it] = pltpu.async_copy(
                table_hbm.at[idx_v.at[pl.ds(it * chunk, chunk)]],
                bufs[b], gsem[b])
            if it >= lag:
                start_wb(it - lag)
        for jt in range(n_iter - lag, n_iter):
            start_wb(jt)
        for jt in range(max(0, n_iter - nbuf), n_iter):
            w[jt].wait()

    return gather_kernel(table, idx)


def _mlp_body(*refs):
    if len(refs) == 11:
        (h1_ref, h2_ref, ea_ref, w1a_ref, w1b_ref, w1c_ref, b1_ref,
         w2_ref, b2_ref, _prev_ref, o_ref) = refs
    else:
        (h1_ref, h2_ref, ea_ref, w1a_ref, w1b_ref, w1c_ref, b1_ref,
         w2_ref, b2_ref, o_ref) = refs
    bf = jnp.bfloat16
    x = jnp.dot(h1_ref[...].astype(bf), w1a_ref[...],
                preferred_element_type=jnp.float32)
    x += jnp.dot(h2_ref[...].astype(bf), w1b_ref[...],
                 preferred_element_type=jnp.float32)
    x += jnp.dot(ea_ref[...].astype(bf), w1c_ref[...],
                 preferred_element_type=jnp.float32)
    x += b1_ref[...]
    xb = x.astype(bf)
    y = (jnp.maximum(xb, 0) + jnp.log1p(jnp.exp(-jnp.abs(xb)))
         - jnp.asarray(LN2, bf))
    o_ref[...] = (
        jnp.dot(y, w2_ref[...], preferred_element_type=jnp.float32)
        + b2_ref[...]
    )


def _tc_mlp_chunk(hh, edge_attr, wb, prev_out, k, n_chunks, block):
    w1a, w1b, w1c, b1r, w2b, b2r = wb
    E = edge_attr.shape[0]
    ec = E // n_chunks
    d_feat = hh.shape[1]
    d_edge = edge_attr.shape[1]
    two_c = w1a.shape[1]
    C = w2b.shape[1]
    nb = ec // block
    koff = k * nb

    in_specs = [
        pl.BlockSpec((block, d_feat), lambda i: (i, 0)),
        pl.BlockSpec((block, d_feat), lambda i: (i + nb, 0)),
        pl.BlockSpec((block, d_edge), lambda i: (koff + i, 0)),
        pl.BlockSpec((d_feat, two_c), lambda i: (0, 0)),
        pl.BlockSpec((d_feat, two_c), lambda i: (0, 0)),
        pl.BlockSpec((d_edge, two_c), lambda i: (0, 0)),
        pl.BlockSpec((1, two_c), lambda i: (0, 0)),
        pl.BlockSpec((two_c, C), lambda i: (0, 0)),
        pl.BlockSpec((1, C), lambda i: (0, 0)),
    ]
    args = [hh, hh, edge_attr, w1a, w1b, w1c, b1r, w2b, b2r]
    aliases = {}
    if prev_out is not None:
        in_specs.append(pl.BlockSpec((8, C), lambda i: (0, 0)))
        args.append(prev_out)
        aliases = {9: 0}

    return pl.pallas_call(
        _mlp_body,
        grid=(nb,),
        in_specs=in_specs,
        out_specs=pl.BlockSpec((block, C), lambda i: (koff + i, 0)),
        out_shape=jax.ShapeDtypeStruct((E, C), jnp.float32),
        input_output_aliases=aliases,
    )(*args)


def kernel(h, edge_attr, edge_index, W1, b1, W2, b2):
    E = edge_attr.shape[0]
    d_feat = h.shape[1]
    d_edge = edge_attr.shape[1]
    two_c = W1.shape[1]
    C = W2.shape[1]
    n_chunks = 2
    ec = E // n_chunks

    ei = edge_index.astype(jnp.int32)
    idx_all = jnp.concatenate(
        [ei[0].reshape(n_chunks, ec), ei[1].reshape(n_chunks, ec)], axis=1)

    wb = (
        W1[:d_feat].astype(jnp.bfloat16),
        W1[d_feat:2 * d_feat].astype(jnp.bfloat16),
        W1[2 * d_feat:].astype(jnp.bfloat16),
        b1.reshape(1, two_c),
        W2.astype(jnp.bfloat16),
        b2.reshape(1, C),
    )

    out = None
    for k in range(n_chunks):
        hh_k = _sc_gather(h, idx_all[k], chunk=200)
        out = _tc_mlp_chunk(edge_attr=edge_attr, hh=hh_k, wb=wb,
                            prev_out=out, k=k, n_chunks=n_chunks, block=4000)
    return out


# speedup vs baseline: 1.0625x; 1.0038x over previous
import functools

import jax
import jax.numpy as jnp
from jax import lax
from jax.experimental import pallas as pl
from jax.experimental.pallas import tpu as pltpu
from jax.experimental.pallas import tpu_sc as plsc

LN2 = 0.6931471805599453


def _sc_gather(table, idx, chunk):
    V, D = table.shape
    B = idx.shape[0]
    mesh = plsc.VectorSubcoreMesh(core_axis_name="c", subcore_axis_name="s")
    nw = 32
    b_per_w = B // nw
    n_iter = b_per_w // chunk
    assert b_per_w % chunk == 0 and chunk % 8 == 0 and n_iter >= 4
    nbuf = 4
    lag = 2

    @functools.partial(
        pl.kernel,
        mesh=mesh,
        out_type=jax.ShapeDtypeStruct((B, D), jnp.float32),
        scratch_types=[
            pltpu.VMEM((b_per_w,), jnp.int32),
        ]
        + [pltpu.VMEM((chunk, D), jnp.float32)] * nbuf
        + [pltpu.SemaphoreType.DMA] * (2 * nbuf),
    )
    def gather_kernel(table_hbm, idx_hbm, out_hbm, idx_v, *bufs_sems):
        bufs = bufs_sems[:nbuf]
        gsem = bufs_sems[nbuf:2 * nbuf]
        wsem = bufs_sems[2 * nbuf:]
        wid = lax.axis_index("s") * 2 + lax.axis_index("c")
        base = wid * b_per_w

        pltpu.sync_copy(idx_hbm.at[pl.ds(base, b_per_w)], idx_v)

        g = {}
        w = {}

        def start_wb(jt):
            g[jt].wait()
            w[jt] = pltpu.async_copy(
                bufs[jt % nbuf],
                out_hbm.at[pl.ds(base + jt * chunk, chunk)],
                wsem[jt % nbuf])

        for it in range(n_iter):
            b = it % nbuf
            if it >= nbuf:
                w[it - nbuf].wait()
            g[it] = pltpu.async_copy(
                table_hbm.at[idx_v.at[pl.ds(it * chunk, chunk)]],
                bufs[b], gsem[b])
            if it >= lag:
                start_wb(it - lag)
        for jt in range(n_iter - lag, n_iter):
            start_wb(jt)
        for jt in range(max(0, n_iter - nbuf), n_iter):
            w[jt].wait()

    return gather_kernel(table, idx)


def _mlp_body(*refs):
    if len(refs) == 11:
        (h1_ref, h2_ref, ea_ref, w1a_ref, w1b_ref, w1c_ref, b1_ref,
         w2_ref, b2_ref, _prev_ref, o_ref) = refs
    else:
        (h1_ref, h2_ref, ea_ref, w1a_ref, w1b_ref, w1c_ref, b1_ref,
         w2_ref, b2_ref, o_ref) = refs
    bf = jnp.bfloat16
    x = jnp.dot(h1_ref[...].astype(bf), w1a_ref[...],
                preferred_element_type=jnp.float32)
    x += jnp.dot(h2_ref[...].astype(bf), w1b_ref[...],
                 preferred_element_type=jnp.float32)
    x += jnp.dot(ea_ref[...].astype(bf), w1c_ref[...],
                 preferred_element_type=jnp.float32)
    x += b1_ref[...]
    xb = x.astype(bf)
    y = (jnp.maximum(xb, 0) + jnp.log1p(jnp.exp(-jnp.abs(xb)))
         - jnp.asarray(LN2, bf))
    o_ref[...] = (
        jnp.dot(y, w2_ref[...], preferred_element_type=jnp.float32)
        + b2_ref[...]
    )


def _tc_mlp_chunk(hh, edge_attr, wb, prev_out, k, n_chunks, block):
    w1a, w1b, w1c, b1r, w2b, b2r = wb
    E = edge_attr.shape[0]
    ec = E // n_chunks
    d_feat = hh.shape[1]
    d_edge = edge_attr.shape[1]
    two_c = w1a.shape[1]
    C = w2b.shape[1]
    nb = ec // block
    koff = k * nb

    in_specs = [
        pl.BlockSpec((block, d_feat), lambda i: (i, 0)),
        pl.BlockSpec((block, d_feat), lambda i: (i + nb, 0)),
        pl.BlockSpec((block, d_edge), lambda i: (koff + i, 0)),
        pl.BlockSpec((d_feat, two_c), lambda i: (0, 0)),
        pl.BlockSpec((d_feat, two_c), lambda i: (0, 0)),
        pl.BlockSpec((d_edge, two_c), lambda i: (0, 0)),
        pl.BlockSpec((1, two_c), lambda i: (0, 0)),
        pl.BlockSpec((two_c, C), lambda i: (0, 0)),
        pl.BlockSpec((1, C), lambda i: (0, 0)),
    ]
    args = [hh, hh, edge_attr, w1a, w1b, w1c, b1r, w2b, b2r]
    aliases = {}
    if prev_out is not None:
        in_specs.append(pl.BlockSpec((8, C), lambda i: (0, 0)))
        args.append(prev_out)
        aliases = {9: 0}

    return pl.pallas_call(
        _mlp_body,
        grid=(nb,),
        in_specs=in_specs,
        out_specs=pl.BlockSpec((block, C), lambda i: (koff + i, 0)),
        out_shape=jax.ShapeDtypeStruct((E, C), jnp.float32),
        input_output_aliases=aliases,
    )(*args)


def kernel(h, edge_attr, edge_index, W1, b1, W2, b2):
    E = edge_attr.shape[0]
    d_feat = h.shape[1]
    d_edge = edge_attr.shape[1]
    two_c = W1.shape[1]
    C = W2.shape[1]
    n_chunks = 4
    ec = E // n_chunks

    ei = edge_index.astype(jnp.int32)
    idx_all = jnp.concatenate(
        [ei[0].reshape(n_chunks, ec), ei[1].reshape(n_chunks, ec)], axis=1)

    wb = (
        W1[:d_feat].astype(jnp.bfloat16),
        W1[d_feat:2 * d_feat].astype(jnp.bfloat16),
        W1[2 * d_feat:].astype(jnp.bfloat16),
        b1.reshape(1, two_c),
        W2.astype(jnp.bfloat16),
        b2.reshape(1, C),
    )

    out = None
    for k in range(n_chunks):
        hh_k = _sc_gather(h, idx_all[k], chunk=200)
        out = _tc_mlp_chunk(edge_attr=edge_attr, hh=hh_k, wb=wb,
                            prev_out=out, k=k, n_chunks=n_chunks, block=4000)
    return out
